# baseline (speedup 1.0000x reference)
import functools

import jax
import jax.numpy as jnp
from jax import lax
from jax.experimental import pallas as pl
from jax.experimental.pallas import tpu as pltpu
from jax.experimental.pallas import tpu_sc as plsc

EMB = 64
CH = 128


def _transform_body(x_ref, w_ref, b_ref, e_ref, out_ref):
    z = lax.dot_general(
        w_ref[...].astype(jnp.bfloat16), x_ref[...].astype(jnp.bfloat16),
        (((1,), (0,)), ((), ())), preferred_element_type=jnp.float32)
    t = jnp.tanh(z + b_ref[...]).astype(jnp.bfloat16)
    out_ref[...] = lax.dot_general(
        t, e_ref[...].astype(jnp.bfloat16), (((0,), (0,)), ((), ())),
        preferred_element_type=jnp.float32)


def _tc_transform(table, W_i, b_i, W_q, b_q):
    v = table.shape[0]
    table_t = table.T
    bk = 8192
    grid = pl.cdiv(v, bk)
    w_stack = jnp.concatenate([W_i.T, W_q.T], axis=0)
    b_stack = jnp.concatenate([b_i, b_q]).reshape(2 * EMB, 1)
    eye = jnp.eye(EMB, dtype=jnp.float32)
    zero = jnp.zeros((EMB, EMB), jnp.float32)
    e_sel = jnp.concatenate(
        [jnp.concatenate([eye, zero], axis=1),
         jnp.concatenate([zero, eye], axis=1)], axis=0)
    out = pl.pallas_call(
        _transform_body,
        grid=(grid,),
        in_specs=[
            pl.BlockSpec((EMB, bk), lambda i: (0, i)),
            pl.BlockSpec((2 * EMB, EMB), lambda i: (0, 0)),
            pl.BlockSpec((2 * EMB, 1), lambda i: (0, 0)),
            pl.BlockSpec((2 * EMB, 2 * EMB), lambda i: (0, 0)),
        ],
        out_specs=pl.BlockSpec((bk, 2 * EMB), lambda i: (i, 0)),
        out_shape=jax.ShapeDtypeStruct((v, 2 * EMB), jnp.float32),
    )(table_t, w_stack, b_stack, e_sel)
    return out.reshape(2 * v, EMB)


def _sc_gather_acc(tfm, idx_t, idx_q, slotmap, zeros, bsz, nw):
    info = plsc.get_sparse_core_info()
    nc, ns = info.num_cores, info.num_subcores
    assert nc * ns == nw
    kt = idx_t.shape[1]
    kq = idx_q.shape[1]

    mesh = plsc.VectorSubcoreMesh(core_axis_name="c", subcore_axis_name="s")

    @functools.partial(
        pl.kernel,
        out_type=(
            jax.ShapeDtypeStruct((nc, bsz, EMB), jnp.float32),
            jax.ShapeDtypeStruct((nc, bsz, EMB), jnp.float32),
        ),
        mesh=mesh,
        compiler_params=pltpu.CompilerParams(use_tc_tiling_on_sc=False),
        scratch_types=[
            pltpu.VMEM((kt, CH), jnp.int32),
            pltpu.VMEM((kq, CH), jnp.int32),
            pltpu.VMEM((8, CH), jnp.int32),
            pltpu.VMEM((CH, EMB), jnp.float32),
            pltpu.VMEM((CH, EMB), jnp.float32),
            pltpu.VMEM((CH, EMB), jnp.float32),
            pltpu.VMEM((CH, EMB), jnp.float32),
            pltpu.VMEM_SHARED((bsz, EMB), jnp.float32),
            pltpu.VMEM_SHARED((bsz, EMB), jnp.float32),
            pltpu.SemaphoreType.DMA,
            pltpu.SemaphoreType.DMA,
            pltpu.SemaphoreType.DMA,
            pltpu.SemaphoreType.DMA,
            pltpu.SemaphoreType.DMA,
            pltpu.SemaphoreType.DMA,
            pltpu.SemaphoreType.DMA,
            pltpu.SemaphoreType.DMA,
        ],
    )
    def k(tfm_h, idxt_h, idxq_h, slot_h, zero_h, out_t_h, out_q_h,
          idxt_v, idxq_v, slot_v, rows0, rows1, rows2, rows3, acc_t, acc_q,
          gsem0, gsem1, gsem2, gsem3, asem0, asem1, asem2, asem3):
        cid = lax.axis_index("c")
        sid = lax.axis_index("s")
        wid = sid * nc + cid
        pltpu.sync_copy(idxt_h.at[wid], idxt_v)
        pltpu.sync_copy(idxq_h.at[wid], idxq_v)
        pltpu.sync_copy(slot_h, slot_v)

        @pl.when(sid == 0)
        def _():
            pltpu.sync_copy(zero_h, acc_t)
            pltpu.sync_copy(zero_h, acc_q)

        plsc.subcore_barrier()

        def run(idx_v, acc, c0, kk):
            assert kk >= 5
            rows = (rows0, rows1, rows2, rows3)
            gsem = (gsem0, gsem1, gsem2, gsem3)
            asem = (asem0, asem1, asem2, asem3)
            for p in range(3):
                pltpu.async_copy(tfm_h.at[idx_v.at[p]], rows[p], gsem[p])

            def step(jj, _):
                for b in range(4):
                    @pl.when(jj % 4 == b)
                    def _():
                        pltpu.make_async_copy(
                            tfm_h.at[idx_v.at[jj]], rows[b], gsem[b]).wait()
                        pltpu.async_copy(
                            rows[b], acc.at[slot_v.at[(c0 + jj) % 8]],
                            asem[b], add=True)

                        @pl.when(jj + 3 < kk)
                        def _():
                            b3 = (b + 3) % 4

                            @pl.when(jj >= 1)
                            def _():
                                pltpu.make_async_copy(
                                    rows[b3],
                                    acc.at[slot_v.at[(c0 + jj - 1) % 8]],
                                    asem[b3]).wait()

                            pltpu.async_copy(
                                tfm_h.at[idx_v.at[jj + 3]], rows[b3],
                                gsem[b3])
                return 0

            lax.fori_loop(0, kk, step, 0)
            for jj in range(kk - 4, kk):
                pltpu.make_async_copy(
                    rows[jj % 4], acc.at[slot_v.at[(c0 + jj) % 8]],
                    asem[jj % 4]).wait()

        run(idxt_v, acc_t, wid * kt, kt)
        run(idxq_v, acc_q, wid * kq, kq)

        plsc.subcore_barrier()

        @pl.when(sid == 0)
        def _():
            pltpu.sync_copy(acc_t, out_t_h.at[cid])
            pltpu.sync_copy(acc_q, out_q_h.at[cid])

    return k(tfm, idx_t, idx_q, slotmap, zeros)


def _mlp_body(st_ref, sq_ref, w1_ref, b1_ref, w2_ref, b2_ref, out_ref,
              *, mt, mq):
    t = (st_ref[0] + st_ref[1]) * (1.0 / mt)
    q = (sq_ref[0] + sq_ref[1]) * (1.0 / mq)
    pool = jnp.concatenate([t, q], axis=-1)
    h = jnp.maximum(pool @ w1_ref[...] + b1_ref[...], 0.0)
    out_ref[...] = h @ w2_ref[...] + b2_ref[...]


def kernel(input_x, input_x_i, input_x_q, table,
           W_t, b_t, W_i, b_i, W_q, b_q, W1, b1, W2, b2):
    del input_x, W_t, b_t
    bsz = input_x_i.shape[0]
    n_t = input_x_i.size
    n_q = input_x_q.size
    mt = n_t // bsz
    mq = n_q // bsz
    nw = 32

    tfm = _tc_transform(table, W_i, b_i, W_q, b_q)

    idx_t = input_x_i.transpose(1, 2, 3, 0).reshape(nw, n_t // nw // CH, CH)
    idx_q = input_x_q.transpose(1, 2, 0).reshape(nw, n_q // nw // CH, CH)
    idx_t2 = idx_t * 2
    idx_q2 = idx_q * 2 + 1

    slotmap = (jnp.arange(8, dtype=jnp.int32)[:, None] * CH
               + jnp.arange(CH, dtype=jnp.int32)[None, :])
    zeros = jnp.zeros((bsz, EMB), jnp.float32)

    sum_t, sum_q = _sc_gather_acc(tfm, idx_t2, idx_q2, slotmap, zeros,
                                  bsz, nw)

    dense = W1.shape[1]
    ncls = W2.shape[1]
    out = pl.pallas_call(
        functools.partial(_mlp_body, mt=mt, mq=mq),
        grid=(1,),
        in_specs=[
            pl.BlockSpec((2, bsz, EMB), lambda i: (0, 0, 0)),
            pl.BlockSpec((2, bsz, EMB), lambda i: (0, 0, 0)),
            pl.BlockSpec((2 * EMB, dense), lambda i: (0, 0)),
            pl.BlockSpec((1, dense), lambda i: (0, 0)),
            pl.BlockSpec((dense, ncls), lambda i: (0, 0)),
            pl.BlockSpec((1, ncls), lambda i: (0, 0)),
        ],
        out_specs=pl.BlockSpec((bsz, ncls), lambda i: (0, 0)),
        out_shape=jax.ShapeDtypeStruct((bsz, ncls), jnp.float32),
    )(sum_t, sum_q, W1, b1.reshape(1, dense), W2, b2.reshape(1, ncls))
    return out


# speedup vs baseline: 1.0737x; 1.0737x over previous
import functools

import jax
import jax.numpy as jnp
from jax import lax
from jax.experimental import pallas as pl
from jax.experimental.pallas import tpu as pltpu
from jax.experimental.pallas import tpu_sc as plsc

EMB = 64
CH = 128


def _transform_body(x_ref, w_ref, b_ref, e_ref, out_ref):
    t = jnp.tanh(w_ref[...] @ x_ref[...] + b_ref[...])
    out_ref[...] = lax.dot_general(
        t, e_ref[...], (((0,), (0,)), ((), ())),
        preferred_element_type=jnp.float32)


def _tc_transform(table, W_i, b_i, W_q, b_q):
    v = table.shape[0]
    table_t = table.T
    bk = 16384
    grid = pl.cdiv(v, bk)
    w_stack = jnp.concatenate([W_i.T, W_q.T], axis=0)
    b_stack = jnp.concatenate([b_i, b_q]).reshape(2 * EMB, 1)
    eye = jnp.eye(EMB, dtype=jnp.float32)
    zero = jnp.zeros((EMB, EMB), jnp.float32)
    e_sel = jnp.concatenate(
        [jnp.concatenate([eye, zero], axis=1),
         jnp.concatenate([zero, eye], axis=1)], axis=0)
    out = pl.pallas_call(
        _transform_body,
        grid=(grid,),
        in_specs=[
            pl.BlockSpec((EMB, bk), lambda i: (0, i)),
            pl.BlockSpec((2 * EMB, EMB), lambda i: (0, 0)),
            pl.BlockSpec((2 * EMB, 1), lambda i: (0, 0)),
            pl.BlockSpec((2 * EMB, 2 * EMB), lambda i: (0, 0)),
        ],
        out_specs=pl.BlockSpec((bk, 2 * EMB), lambda i: (i, 0)),
        out_shape=jax.ShapeDtypeStruct((v, 2 * EMB), jnp.float32),
    )(table_t, w_stack, b_stack, e_sel)
    return out.reshape(2 * v, EMB)


def _sc_gather_acc(tfm, idx_t, idx_q, slotmap, zeros, bsz, nw):
    info = plsc.get_sparse_core_info()
    nc, ns = info.num_cores, info.num_subcores
    assert nc * ns == nw
    kt = idx_t.shape[1]
    kq = idx_q.shape[1]

    mesh = plsc.VectorSubcoreMesh(core_axis_name="c", subcore_axis_name="s")

    @functools.partial(
        pl.kernel,
        out_type=(
            jax.ShapeDtypeStruct((nc, bsz, EMB), jnp.float32),
            jax.ShapeDtypeStruct((nc, bsz, EMB), jnp.float32),
        ),
        mesh=mesh,
        compiler_params=pltpu.CompilerParams(use_tc_tiling_on_sc=False),
        scratch_types=[
            pltpu.VMEM((kt, CH), jnp.int32),
            pltpu.VMEM((kq, CH), jnp.int32),
            pltpu.VMEM((8, CH), jnp.int32),
            pltpu.VMEM((CH, EMB), jnp.float32),
            pltpu.VMEM((CH, EMB), jnp.float32),
            pltpu.VMEM((CH, EMB), jnp.float32),
            pltpu.VMEM((CH, EMB), jnp.float32),
            pltpu.VMEM_SHARED((bsz, EMB), jnp.float32),
            pltpu.VMEM_SHARED((bsz, EMB), jnp.float32),
            pltpu.SemaphoreType.DMA,
            pltpu.SemaphoreType.DMA,
            pltpu.SemaphoreType.DMA,
            pltpu.SemaphoreType.DMA,
            pltpu.SemaphoreType.DMA,
            pltpu.SemaphoreType.DMA,
            pltpu.SemaphoreType.DMA,
            pltpu.SemaphoreType.DMA,
        ],
    )
    def k(tfm_h, idxt_h, idxq_h, slot_h, zero_h, out_t_h, out_q_h,
          idxt_v, idxq_v, slot_v, rows0, rows1, rows2, rows3, acc_t, acc_q,
          gsem0, gsem1, gsem2, gsem3, asem0, asem1, asem2, asem3):
        cid = lax.axis_index("c")
        sid = lax.axis_index("s")
        wid = sid * nc + cid
        pltpu.sync_copy(idxt_h.at[wid], idxt_v)
        pltpu.sync_copy(idxq_h.at[wid], idxq_v)
        pltpu.sync_copy(slot_h, slot_v)

        @pl.when(sid == 0)
        def _():
            pltpu.sync_copy(zero_h, acc_t)
            pltpu.sync_copy(zero_h, acc_q)

        plsc.subcore_barrier()

        def run(idx_v, acc, c0, kk):
            assert kk >= 5
            rows = (rows0, rows1, rows2, rows3)
            gsem = (gsem0, gsem1, gsem2, gsem3)
            asem = (asem0, asem1, asem2, asem3)
            for p in range(3):
                pltpu.async_copy(tfm_h.at[idx_v.at[p]], rows[p], gsem[p])

            def step(jj, _):
                for b in range(4):
                    @pl.when(jj % 4 == b)
                    def _():
                        pltpu.make_async_copy(
                            tfm_h.at[idx_v.at[jj]], rows[b], gsem[b]).wait()
                        pltpu.async_copy(
                            rows[b], acc.at[slot_v.at[(c0 + jj) % 8]],
                            asem[b], add=True)

                        @pl.when(jj + 3 < kk)
                        def _():
                            b3 = (b + 3) % 4

                            @pl.when(jj >= 1)
                            def _():
                                pltpu.make_async_copy(
                                    rows[b3],
                                    acc.at[slot_v.at[(c0 + jj - 1) % 8]],
                                    asem[b3]).wait()

                            pltpu.async_copy(
                                tfm_h.at[idx_v.at[jj + 3]], rows[b3],
                                gsem[b3])
                return 0

            lax.fori_loop(0, kk, step, 0)
            for jj in range(kk - 4, kk):
                pltpu.make_async_copy(
                    rows[jj % 4], acc.at[slot_v.at[(c0 + jj) % 8]],
                    asem[jj % 4]).wait()

        run(idxt_v, acc_t, wid * kt, kt)
        run(idxq_v, acc_q, wid * kq, kq)

        plsc.subcore_barrier()

        @pl.when(sid == 0)
        def _():
            pltpu.sync_copy(acc_t, out_t_h.at[cid])
            pltpu.sync_copy(acc_q, out_q_h.at[cid])

    return k(tfm, idx_t, idx_q, slotmap, zeros)


def _mlp_body(st_ref, sq_ref, w1_ref, b1_ref, w2_ref, b2_ref, out_ref,
              *, mt, mq):
    t = (st_ref[0] + st_ref[1]) * (1.0 / mt)
    q = (sq_ref[0] + sq_ref[1]) * (1.0 / mq)
    pool = jnp.concatenate([t, q], axis=-1)
    h = jnp.maximum(pool @ w1_ref[...] + b1_ref[...], 0.0)
    out_ref[...] = h @ w2_ref[...] + b2_ref[...]


def kernel(input_x, input_x_i, input_x_q, table,
           W_t, b_t, W_i, b_i, W_q, b_q, W1, b1, W2, b2):
    del input_x, W_t, b_t
    bsz = input_x_i.shape[0]
    n_t = input_x_i.size
    n_q = input_x_q.size
    mt = n_t // bsz
    mq = n_q // bsz
    nw = 32

    tfm = _tc_transform(table, W_i, b_i, W_q, b_q)

    idx_t = input_x_i.transpose(1, 2, 3, 0).reshape(nw, n_t // nw // CH, CH)
    idx_q = input_x_q.transpose(1, 2, 0).reshape(nw, n_q // nw // CH, CH)
    idx_t2 = idx_t * 2
    idx_q2 = idx_q * 2 + 1

    slotmap = (jnp.arange(8, dtype=jnp.int32)[:, None] * CH
               + jnp.arange(CH, dtype=jnp.int32)[None, :])
    zeros = jnp.zeros((bsz, EMB), jnp.float32)

    sum_t, sum_q = _sc_gather_acc(tfm, idx_t2, idx_q2, slotmap, zeros,
                                  bsz, nw)

    dense = W1.shape[1]
    ncls = W2.shape[1]
    out = pl.pallas_call(
        functools.partial(_mlp_body, mt=mt, mq=mq),
        grid=(1,),
        in_specs=[
            pl.BlockSpec((2, bsz, EMB), lambda i: (0, 0, 0)),
            pl.BlockSpec((2, bsz, EMB), lambda i: (0, 0, 0)),
            pl.BlockSpec((2 * EMB, dense), lambda i: (0, 0)),
            pl.BlockSpec((1, dense), lambda i: (0, 0)),
            pl.BlockSpec((dense, ncls), lambda i: (0, 0)),
            pl.BlockSpec((1, ncls), lambda i: (0, 0)),
        ],
        out_specs=pl.BlockSpec((bsz, ncls), lambda i: (0, 0)),
        out_shape=jax.ShapeDtypeStruct((bsz, ncls), jnp.float32),
    )(sum_t, sum_q, W1, b1.reshape(1, dense), W2, b2.reshape(1, ncls))
    return out


# speedup vs baseline: 1.0894x; 1.0146x over previous
import functools

import jax
import jax.numpy as jnp
from jax import lax
from jax.experimental import pallas as pl
from jax.experimental.pallas import tpu as pltpu
from jax.experimental.pallas import tpu_sc as plsc

EMB = 64
CH = 128


def _transform_body(x_ref, w_ref, b_ref, e_ref, out_ref):
    t = jnp.tanh(w_ref[...] @ x_ref[...] + b_ref[...])
    out_ref[...] = lax.dot_general(
        t, e_ref[...], (((0,), (0,)), ((), ())),
        preferred_element_type=jnp.float32)


def _tc_transform(table, W_i, b_i, W_q, b_q):
    v = table.shape[0]
    table_t = table.T
    bk = 32768
    grid = pl.cdiv(v, bk)
    w_stack = jnp.concatenate([W_i.T, W_q.T], axis=0)
    b_stack = jnp.concatenate([b_i, b_q]).reshape(2 * EMB, 1)
    eye = jnp.eye(EMB, dtype=jnp.float32)
    zero = jnp.zeros((EMB, EMB), jnp.float32)
    e_sel = jnp.concatenate(
        [jnp.concatenate([eye, zero], axis=1),
         jnp.concatenate([zero, eye], axis=1)], axis=0)
    out = pl.pallas_call(
        _transform_body,
        grid=(grid,),
        in_specs=[
            pl.BlockSpec((EMB, bk), lambda i: (0, i)),
            pl.BlockSpec((2 * EMB, EMB), lambda i: (0, 0)),
            pl.BlockSpec((2 * EMB, 1), lambda i: (0, 0)),
            pl.BlockSpec((2 * EMB, 2 * EMB), lambda i: (0, 0)),
        ],
        out_specs=pl.BlockSpec((bk, 2 * EMB), lambda i: (i, 0)),
        out_shape=jax.ShapeDtypeStruct((v, 2 * EMB), jnp.float32),
    )(table_t, w_stack, b_stack, e_sel)
    return out.reshape(2 * v, EMB)


def _sc_gather_acc(tfm, idx_t, idx_q, slotmap, zeros, bsz, nw):
    info = plsc.get_sparse_core_info()
    nc, ns = info.num_cores, info.num_subcores
    assert nc * ns == nw
    kt = idx_t.shape[1]
    kq = idx_q.shape[1]

    mesh = plsc.VectorSubcoreMesh(core_axis_name="c", subcore_axis_name="s")

    @functools.partial(
        pl.kernel,
        out_type=(
            jax.ShapeDtypeStruct((nc, bsz, EMB), jnp.float32),
            jax.ShapeDtypeStruct((nc, bsz, EMB), jnp.float32),
        ),
        mesh=mesh,
        compiler_params=pltpu.CompilerParams(use_tc_tiling_on_sc=False),
        scratch_types=[
            pltpu.VMEM((kt, CH), jnp.int32),
            pltpu.VMEM((kq, CH), jnp.int32),
            pltpu.VMEM((8, CH), jnp.int32),
            pltpu.VMEM((CH, EMB), jnp.float32),
            pltpu.VMEM((CH, EMB), jnp.float32),
            pltpu.VMEM((CH, EMB), jnp.float32),
            pltpu.VMEM((CH, EMB), jnp.float32),
            pltpu.VMEM_SHARED((bsz, EMB), jnp.float32),
            pltpu.VMEM_SHARED((bsz, EMB), jnp.float32),
            pltpu.SemaphoreType.DMA,
            pltpu.SemaphoreType.DMA,
            pltpu.SemaphoreType.DMA,
            pltpu.SemaphoreType.DMA,
            pltpu.SemaphoreType.DMA,
            pltpu.SemaphoreType.DMA,
            pltpu.SemaphoreType.DMA,
            pltpu.SemaphoreType.DMA,
        ],
    )
    def k(tfm_h, idxt_h, idxq_h, slot_h, zero_h, out_t_h, out_q_h,
          idxt_v, idxq_v, slot_v, rows0, rows1, rows2, rows3, acc_t, acc_q,
          gsem0, gsem1, gsem2, gsem3, asem0, asem1, asem2, asem3):
        cid = lax.axis_index("c")
        sid = lax.axis_index("s")
        wid = sid * nc + cid
        pltpu.sync_copy(idxt_h.at[wid], idxt_v)
        pltpu.sync_copy(idxq_h.at[wid], idxq_v)
        pltpu.sync_copy(slot_h, slot_v)

        @pl.when(sid == 0)
        def _():
            pltpu.sync_copy(zero_h, acc_t)
            pltpu.sync_copy(zero_h, acc_q)

        plsc.subcore_barrier()

        def run(idx_v, acc, c0, kk):
            assert kk >= 5
            rows = (rows0, rows1, rows2, rows3)
            gsem = (gsem0, gsem1, gsem2, gsem3)
            asem = (asem0, asem1, asem2, asem3)
            for p in range(3):
                pltpu.async_copy(tfm_h.at[idx_v.at[p]], rows[p], gsem[p])

            def step(jj, _):
                for b in range(4):
                    @pl.when(jj % 4 == b)
                    def _():
                        pltpu.make_async_copy(
                            tfm_h.at[idx_v.at[jj]], rows[b], gsem[b]).wait()
                        pltpu.async_copy(
                            rows[b], acc.at[slot_v.at[(c0 + jj) % 8]],
                            asem[b], add=True)

                        @pl.when(jj + 3 < kk)
                        def _():
                            b3 = (b + 3) % 4

                            @pl.when(jj >= 1)
                            def _():
                                pltpu.make_async_copy(
                                    rows[b3],
                                    acc.at[slot_v.at[(c0 + jj - 1) % 8]],
                                    asem[b3]).wait()

                            pltpu.async_copy(
                                tfm_h.at[idx_v.at[jj + 3]], rows[b3],
                                gsem[b3])
                return 0

            lax.fori_loop(0, kk, step, 0)
            for jj in range(kk - 4, kk):
                pltpu.make_async_copy(
                    rows[jj % 4], acc.at[slot_v.at[(c0 + jj) % 8]],
                    asem[jj % 4]).wait()

        run(idxt_v, acc_t, wid * kt, kt)
        run(idxq_v, acc_q, wid * kq, kq)

        plsc.subcore_barrier()

        @pl.when(sid == 0)
        def _():
            pltpu.sync_copy(acc_t, out_t_h.at[cid])
            pltpu.sync_copy(acc_q, out_q_h.at[cid])

    return k(tfm, idx_t, idx_q, slotmap, zeros)


def _mlp_body(st_ref, sq_ref, w1_ref, b1_ref, w2_ref, b2_ref, out_ref,
              *, mt, mq):
    t = (st_ref[0] + st_ref[1]) * (1.0 / mt)
    q = (sq_ref[0] + sq_ref[1]) * (1.0 / mq)
    pool = jnp.concatenate([t, q], axis=-1)
    h = jnp.maximum(pool @ w1_ref[...] + b1_ref[...], 0.0)
    out_ref[...] = h @ w2_ref[...] + b2_ref[...]


def kernel(input_x, input_x_i, input_x_q, table,
           W_t, b_t, W_i, b_i, W_q, b_q, W1, b1, W2, b2):
    del input_x, W_t, b_t
    bsz = input_x_i.shape[0]
    n_t = input_x_i.size
    n_q = input_x_q.size
    mt = n_t // bsz
    mq = n_q // bsz
    nw = 32

    tfm = _tc_transform(table, W_i, b_i, W_q, b_q)

    idx_t = input_x_i.transpose(1, 2, 3, 0).reshape(nw, n_t // nw // CH, CH)
    idx_q = input_x_q.transpose(1, 2, 0).reshape(nw, n_q // nw // CH, CH)
    idx_t2 = idx_t * 2
    idx_q2 = idx_q * 2 + 1

    slotmap = (jnp.arange(8, dtype=jnp.int32)[:, None] * CH
               + jnp.arange(CH, dtype=jnp.int32)[None, :])
    zeros = jnp.zeros((bsz, EMB), jnp.float32)

    sum_t, sum_q = _sc_gather_acc(tfm, idx_t2, idx_q2, slotmap, zeros,
                                  bsz, nw)

    dense = W1.shape[1]
    ncls = W2.shape[1]
    out = pl.pallas_call(
        functools.partial(_mlp_body, mt=mt, mq=mq),
        grid=(1,),
        in_specs=[
            pl.BlockSpec((2, bsz, EMB), lambda i: (0, 0, 0)),
            pl.BlockSpec((2, bsz, EMB), lambda i: (0, 0, 0)),
            pl.BlockSpec((2 * EMB, dense), lambda i: (0, 0)),
            pl.BlockSpec((1, dense), lambda i: (0, 0)),
            pl.BlockSpec((dense, ncls), lambda i: (0, 0)),
            pl.BlockSpec((1, ncls), lambda i: (0, 0)),
        ],
        out_specs=pl.BlockSpec((bsz, ncls), lambda i: (0, 0)),
        out_shape=jax.ShapeDtypeStruct((bsz, ncls), jnp.float32),
    )(sum_t, sum_q, W1, b1.reshape(1, dense), W2, b2.reshape(1, ncls))
    return out


# speedup vs baseline: 1.1247x; 1.0324x over previous
import functools

import jax
import jax.numpy as jnp
from jax import lax
from jax.experimental import pallas as pl
from jax.experimental.pallas import tpu as pltpu
from jax.experimental.pallas import tpu_sc as plsc

EMB = 64
CH = 128


def _transform_body(x_ref, w_ref, b_ref, e_ref, out_ref):
    t = jnp.tanh(w_ref[...] @ x_ref[...] + b_ref[...])
    out_ref[...] = lax.dot_general(
        t, e_ref[...], (((0,), (0,)), ((), ())),
        preferred_element_type=jnp.float32)


def _tc_transform(table, W_i, b_i, W_q, b_q):
    v = table.shape[0]
    table_t = table.T
    bk = 32768
    grid = pl.cdiv(v, bk)
    w_stack = jnp.concatenate([W_i.T, W_q.T], axis=0)
    b_stack = jnp.concatenate([b_i, b_q]).reshape(2 * EMB, 1)
    eye = jnp.eye(EMB, dtype=jnp.float32)
    zero = jnp.zeros((EMB, EMB), jnp.float32)
    e_sel = jnp.concatenate(
        [jnp.concatenate([eye, zero], axis=1),
         jnp.concatenate([zero, eye], axis=1)], axis=0)
    out = pl.pallas_call(
        _transform_body,
        grid=(grid,),
        in_specs=[
            pl.BlockSpec((EMB, bk), lambda i: (0, i)),
            pl.BlockSpec((2 * EMB, EMB), lambda i: (0, 0)),
            pl.BlockSpec((2 * EMB, 1), lambda i: (0, 0)),
            pl.BlockSpec((2 * EMB, 2 * EMB), lambda i: (0, 0)),
        ],
        out_specs=pl.BlockSpec((bk, 2 * EMB), lambda i: (i, 0)),
        out_shape=jax.ShapeDtypeStruct((v, 2 * EMB), jnp.float32),
    )(table_t, w_stack, b_stack, e_sel)
    return out.reshape(2 * v, EMB)


def _sc_gather_acc(tfm, idx_t, idx_q, slotmap, zeros, bsz, nw):
    info = plsc.get_sparse_core_info()
    nc, ns = info.num_cores, info.num_subcores
    assert nc * ns == nw
    kt = idx_t.shape[1]
    kq = idx_q.shape[1]

    mesh = plsc.VectorSubcoreMesh(core_axis_name="c", subcore_axis_name="s")

    @functools.partial(
        pl.kernel,
        out_type=(
            jax.ShapeDtypeStruct((nc, bsz, EMB), jnp.float32),
            jax.ShapeDtypeStruct((nc, bsz, EMB), jnp.float32),
        ),
        mesh=mesh,
        compiler_params=pltpu.CompilerParams(use_tc_tiling_on_sc=False),
        scratch_types=[
            pltpu.VMEM((kt, CH), jnp.int32),
            pltpu.VMEM((kq, CH), jnp.int32),
            pltpu.VMEM((8, CH), jnp.int32),
            pltpu.VMEM((CH, EMB), jnp.float32),
            pltpu.VMEM((CH, EMB), jnp.float32),
            pltpu.VMEM((CH, EMB), jnp.float32),
            pltpu.VMEM((CH, EMB), jnp.float32),
            pltpu.VMEM((CH, EMB), jnp.float32),
            pltpu.VMEM((CH, EMB), jnp.float32),
            pltpu.VMEM_SHARED((bsz, EMB), jnp.float32),
            pltpu.VMEM_SHARED((bsz, EMB), jnp.float32),
        ] + [pltpu.SemaphoreType.DMA] * 12,
    )
    def k(tfm_h, idxt_h, idxq_h, slot_h, zero_h, out_t_h, out_q_h,
          idxt_v, idxq_v, slot_v, rows0, rows1, rows2, rows3, rows4, rows5,
          acc_t, acc_q, gsem0, gsem1, gsem2, gsem3, gsem4, gsem5,
          asem0, asem1, asem2, asem3, asem4, asem5):
        cid = lax.axis_index("c")
        sid = lax.axis_index("s")
        wid = sid * nc + cid
        pltpu.sync_copy(idxt_h.at[wid], idxt_v)
        pltpu.sync_copy(idxq_h.at[wid], idxq_v)
        pltpu.sync_copy(slot_h, slot_v)

        @pl.when(sid == 0)
        def _():
            pltpu.sync_copy(zero_h, acc_t)
            pltpu.sync_copy(zero_h, acc_q)

        plsc.subcore_barrier()

        def run(idx_v, acc, c0, kk):
            nb = 6
            assert kk >= nb + 1
            rows = (rows0, rows1, rows2, rows3, rows4, rows5)
            gsem = (gsem0, gsem1, gsem2, gsem3, gsem4, gsem5)
            asem = (asem0, asem1, asem2, asem3, asem4, asem5)
            for p in range(nb - 1):
                pltpu.async_copy(tfm_h.at[idx_v.at[p]], rows[p], gsem[p])

            def step(jj, _):
                for b in range(nb):
                    @pl.when(jj % nb == b)
                    def _():
                        pltpu.make_async_copy(
                            tfm_h.at[idx_v.at[jj]], rows[b], gsem[b]).wait()
                        pltpu.async_copy(
                            rows[b], acc.at[slot_v.at[(c0 + jj) % 8]],
                            asem[b], add=True)

                        @pl.when(jj + nb - 1 < kk)
                        def _():
                            bn = (b + nb - 1) % nb

                            @pl.when(jj >= 1)
                            def _():
                                pltpu.make_async_copy(
                                    rows[bn],
                                    acc.at[slot_v.at[(c0 + jj - 1) % 8]],
                                    asem[bn]).wait()

                            pltpu.async_copy(
                                tfm_h.at[idx_v.at[jj + nb - 1]], rows[bn],
                                gsem[bn])
                return 0

            lax.fori_loop(0, kk, step, 0)
            for jj in range(kk - nb, kk):
                pltpu.make_async_copy(
                    rows[jj % nb], acc.at[slot_v.at[(c0 + jj) % 8]],
                    asem[jj % nb]).wait()

        run(idxt_v, acc_t, wid * kt, kt)
        run(idxq_v, acc_q, wid * kq, kq)

        plsc.subcore_barrier()

        @pl.when(sid == 0)
        def _():
            pltpu.sync_copy(acc_t, out_t_h.at[cid])
            pltpu.sync_copy(acc_q, out_q_h.at[cid])

    return k(tfm, idx_t, idx_q, slotmap, zeros)


def _mlp_body(st_ref, sq_ref, w1_ref, b1_ref, w2_ref, b2_ref, out_ref,
              *, mt, mq):
    t = (st_ref[0] + st_ref[1]) * (1.0 / mt)
    q = (sq_ref[0] + sq_ref[1]) * (1.0 / mq)
    pool = jnp.concatenate([t, q], axis=-1)
    h = jnp.maximum(pool @ w1_ref[...] + b1_ref[...], 0.0)
    out_ref[...] = h @ w2_ref[...] + b2_ref[...]


def kernel(input_x, input_x_i, input_x_q, table,
           W_t, b_t, W_i, b_i, W_q, b_q, W1, b1, W2, b2):
    del input_x, W_t, b_t
    bsz = input_x_i.shape[0]
    n_t = input_x_i.size
    n_q = input_x_q.size
    mt = n_t // bsz
    mq = n_q // bsz
    nw = 32

    tfm = _tc_transform(table, W_i, b_i, W_q, b_q)

    idx_t = input_x_i.transpose(1, 2, 3, 0).reshape(nw, n_t // nw // CH, CH)
    idx_q = input_x_q.transpose(1, 2, 0).reshape(nw, n_q // nw // CH, CH)
    idx_t2 = idx_t * 2
    idx_q2 = idx_q * 2 + 1

    slotmap = (jnp.arange(8, dtype=jnp.int32)[:, None] * CH
               + jnp.arange(CH, dtype=jnp.int32)[None, :])
    zeros = jnp.zeros((bsz, EMB), jnp.float32)

    sum_t, sum_q = _sc_gather_acc(tfm, idx_t2, idx_q2, slotmap, zeros,
                                  bsz, nw)

    dense = W1.shape[1]
    ncls = W2.shape[1]
    out = pl.pallas_call(
        functools.partial(_mlp_body, mt=mt, mq=mq),
        grid=(1,),
        in_specs=[
            pl.BlockSpec((2, bsz, EMB), lambda i: (0, 0, 0)),
            pl.BlockSpec((2, bsz, EMB), lambda i: (0, 0, 0)),
            pl.BlockSpec((2 * EMB, dense), lambda i: (0, 0)),
            pl.BlockSpec((1, dense), lambda i: (0, 0)),
            pl.BlockSpec((dense, ncls), lambda i: (0, 0)),
            pl.BlockSpec((1, ncls), lambda i: (0, 0)),
        ],
        out_specs=pl.BlockSpec((bsz, ncls), lambda i: (0, 0)),
        out_shape=jax.ShapeDtypeStruct((bsz, ncls), jnp.float32),
    )(sum_t, sum_q, W1, b1.reshape(1, dense), W2, b2.reshape(1, ncls))
    return out
